# baseline (speedup 1.0000x reference)
import jax
import jax.numpy as jnp
from jax import lax
from jax.experimental import pallas as pl
from jax.experimental.pallas import tpu as pltpu
from jax.experimental.pallas import tpu_sc as plsc

VOCAB = 1000000
EMBED_DIM = 64
PAD_DIM = 128
BATCH = 4096
SEQ = 200

NUM_CORES = 2
NUM_SUBCORES = 16
NUM_WORKERS = NUM_CORES * NUM_SUBCORES
IDX_PER_WORKER = BATCH * SEQ // NUM_WORKERS
ROWS_PER_WORKER = BATCH // NUM_WORKERS
BLOCK = 128
NUM_BLOCKS = IDX_PER_WORKER // BLOCK
CORE_ROWS = NUM_SUBCORES * ROWS_PER_WORKER
ACC_ROWS = CORE_ROWS + NUM_SUBCORES

TP_CHUNK = 4096
SPLIT = 499712
HALF_ROWS = (SPLIT, VOCAB - SPLIT)
HALF_BLOCK0 = (0, SPLIT // TP_CHUNK)


def _transpose_pad_kernel(in_ref, out_ref):
    out_ref[:, :EMBED_DIM] = in_ref[...].T


def _transpose_pad(table_t, half):
    rows = HALF_ROWS[half]
    block0 = HALF_BLOCK0[half]
    grid = (pl.cdiv(rows, TP_CHUNK),)
    return pl.pallas_call(
        _transpose_pad_kernel,
        grid=grid,
        in_specs=[pl.BlockSpec((EMBED_DIM, TP_CHUNK),
                               lambda j: (0, j + block0))],
        out_specs=pl.BlockSpec((TP_CHUNK, PAD_DIM), lambda j: (j, 0)),
        out_shape=jax.ShapeDtypeStruct((rows, PAD_DIM), jnp.float32),
    )(table_t)


def _sc_pool_kernel(idx_hbm, scat_hbm, table_hbm, out_hbm,
                    idx_v, scat_v, rows_v, acc_sh, gsems, ssems):
    cid = lax.axis_index("c")
    sid = lax.axis_index("s")
    wid = sid * NUM_CORES + cid

    pltpu.sync_copy(idx_hbm.at[wid], idx_v)
    pltpu.sync_copy(scat_hbm.at[wid], scat_v)

    zeros16 = jnp.zeros((16,), jnp.float32)

    @pl.loop(0, BLOCK)
    def _(r):
        @pl.loop(0, EMBED_DIM, step=16)
        def _(c):
            rows_v[0, r, pl.ds(c, 16)] = zeros16

    pltpu.sync_copy(rows_v.at[0],
                    acc_sh.at[pl.ds(sid * ROWS_PER_WORKER, ROWS_PER_WORKER)])

    def gather(b, slot):
        return pltpu.make_async_copy(
            table_hbm.at[idx_v.at[b]], rows_v.at[slot], gsems.at[slot])

    def scatter_start(b, slot):
        pltpu.async_copy(
            rows_v.at[slot], acc_sh.at[scat_v.at[b]], ssems.at[slot],
            add=True)

    def scatter_wait(b, slot):
        pltpu.make_async_copy(
            rows_v.at[slot], acc_sh.at[scat_v.at[b]], ssems.at[slot]).wait()

    gather(0, 0).start()
    gather(1, 1).start()

    @pl.loop(0, NUM_BLOCKS, step=2)
    def _(b):
        gather(b, 0).wait()
        scatter_start(b, 0)
        gather(b + 1, 1).wait()
        scatter_start(b + 1, 1)
        scatter_wait(b, 0)

        @pl.when(b + 2 < NUM_BLOCKS)
        def _():
            gather(b + 2, 0).start()

        scatter_wait(b + 1, 1)

        @pl.when(b + 3 < NUM_BLOCKS)
        def _():
            gather(b + 3, 1).start()

    pltpu.sync_copy(acc_sh.at[pl.ds(sid * ROWS_PER_WORKER, ROWS_PER_WORKER)],
                    out_hbm.at[pl.ds(wid * ROWS_PER_WORKER, ROWS_PER_WORKER)])


def _make_pool(rows):
    mesh = plsc.VectorSubcoreMesh(core_axis_name="c", subcore_axis_name="s")
    return pl.kernel(
        _sc_pool_kernel,
        out_type=jax.ShapeDtypeStruct((BATCH, EMBED_DIM), jnp.float32),
        mesh=mesh,
        compiler_params=pltpu.CompilerParams(use_tc_tiling_on_sc=False),
        scratch_types=[
            pltpu.VMEM((NUM_BLOCKS, BLOCK), jnp.int32),
            pltpu.VMEM((NUM_BLOCKS, BLOCK), jnp.int32),
            pltpu.VMEM((2, BLOCK, EMBED_DIM), jnp.float32),
            pltpu.VMEM_SHARED((ACC_ROWS, EMBED_DIM), jnp.float32),
            pltpu.SemaphoreType.DMA((2,)),
            pltpu.SemaphoreType.DMA((2,)),
        ],
    )


@jax.jit
def _pooled_halves(input_text, table):
    flat = input_text.reshape(NUM_WORKERS, NUM_BLOCKS, BLOCK)
    base = (jnp.arange(IDX_PER_WORKER, dtype=jnp.int32) // SEQ).reshape(
        1, NUM_BLOCKS, BLOCK)
    offs = ((jnp.arange(NUM_WORKERS, dtype=jnp.int32) // NUM_CORES)
            * ROWS_PER_WORKER).reshape(NUM_WORKERS, 1, 1)
    scat_real = base + offs
    junk = (CORE_ROWS
            + (jnp.arange(NUM_WORKERS, dtype=jnp.int32) // NUM_CORES)
            .reshape(NUM_WORKERS, 1, 1))

    in_lo = flat < SPLIT
    idx_lo = jnp.where(in_lo, flat * 2, 0)
    scat_lo = jnp.where(in_lo, scat_real, junk)
    idx_hi = jnp.where(in_lo, 0, (flat - SPLIT) * 2)
    scat_hi = jnp.where(in_lo, junk, scat_real)

    table_t = table.T
    lin_lo = _transpose_pad(table_t, 0).reshape(2 * HALF_ROWS[0], EMBED_DIM)
    pooled_lo = _make_pool(HALF_ROWS[0])(idx_lo, scat_lo, lin_lo)
    lin_hi = _transpose_pad(table_t, 1).reshape(2 * HALF_ROWS[1], EMBED_DIM)
    pooled_hi = _make_pool(HALF_ROWS[1])(idx_hi, scat_hi, lin_hi)
    return pooled_lo, pooled_hi


def _mlp_kernel(xa_ref, xb_ref, len_ref, w1_ref, b1_ref, w2_ref, b2_ref,
                out_ref):
    x = (xa_ref[...] + xb_ref[...]) / len_ref[...]
    h = jnp.maximum(
        jnp.dot(x, w1_ref[...], preferred_element_type=jnp.float32)
        + b1_ref[...], 0.0)
    out_ref[...] = (
        jnp.dot(h, w2_ref[...], preferred_element_type=jnp.float32)
        + b2_ref[...])


@jax.jit
def _mlp(pooled_lo, pooled_hi, text_len, W1, b1, W2, b2):
    bm = 512
    n_hidden = W1.shape[1]
    n_classes = W2.shape[1]
    grid = (BATCH // bm,)
    return pl.pallas_call(
        _mlp_kernel,
        grid=grid,
        in_specs=[
            pl.BlockSpec((bm, EMBED_DIM), lambda i: (i, 0)),
            pl.BlockSpec((bm, EMBED_DIM), lambda i: (i, 0)),
            pl.BlockSpec((bm, 1), lambda i: (i, 0)),
            pl.BlockSpec((EMBED_DIM, n_hidden), lambda i: (0, 0)),
            pl.BlockSpec((1, n_hidden), lambda i: (0, 0)),
            pl.BlockSpec((n_hidden, n_classes), lambda i: (0, 0)),
            pl.BlockSpec((1, n_classes), lambda i: (0, 0)),
        ],
        out_specs=pl.BlockSpec((bm, n_classes), lambda i: (i, 0)),
        out_shape=jax.ShapeDtypeStruct((BATCH, n_classes), jnp.float32),
    )(pooled_lo, pooled_hi, text_len.reshape(BATCH, 1), W1,
      b1.reshape(1, n_hidden), W2, b2.reshape(1, n_classes))


def kernel(input_text, text_len, table, W1, b1, W2, b2):
    pooled_lo, pooled_hi = _pooled_halves(input_text, table)
    return _mlp(pooled_lo, pooled_hi, text_len, W1, b1, W2, b2)


# speedup vs baseline: 19.4874x; 19.4874x over previous
import jax
import jax.numpy as jnp
from jax import lax
from jax.experimental import pallas as pl
from jax.experimental.pallas import tpu as pltpu
from jax.experimental.pallas import tpu_sc as plsc

VOCAB = 1000000
EMBED_DIM = 64
PAD_DIM = 128
BATCH = 4096
SEQ = 200

NUM_CORES = 2
NUM_SUBCORES = 16
NUM_WORKERS = NUM_CORES * NUM_SUBCORES
IDX_PER_WORKER = BATCH * SEQ // NUM_WORKERS
ROWS_PER_WORKER = BATCH // NUM_WORKERS
BLOCK = 128
NUM_BLOCKS = IDX_PER_WORKER // BLOCK
CORE_ROWS = NUM_SUBCORES * ROWS_PER_WORKER
ACC_ROWS = 2 * CORE_ROWS

TP_CHUNK = 4096
SPLIT = 499712
HALF_ROWS = (SPLIT, VOCAB - SPLIT)
HALF_BLOCK0 = (0, SPLIT // TP_CHUNK)


def _transpose_pad_kernel(in_ref, out_ref):
    out_ref[:, :EMBED_DIM] = in_ref[...].T


def _transpose_pad(table_t, half):
    rows = HALF_ROWS[half]
    block0 = HALF_BLOCK0[half]
    grid = (pl.cdiv(rows, TP_CHUNK),)
    return pl.pallas_call(
        _transpose_pad_kernel,
        grid=grid,
        in_specs=[pl.BlockSpec((EMBED_DIM, TP_CHUNK),
                               lambda j: (0, j + block0))],
        out_specs=pl.BlockSpec((TP_CHUNK, PAD_DIM), lambda j: (j, 0)),
        out_shape=jax.ShapeDtypeStruct((rows, PAD_DIM), jnp.float32),
    )(table_t)


def _sc_pool_kernel(idx_hbm, scat_hbm, table_hbm, out_hbm,
                    idx_v, scat_v, rows_v, acc_sh, gsems, ssems):
    cid = lax.axis_index("c")
    sid = lax.axis_index("s")
    wid = sid * NUM_CORES + cid

    pltpu.sync_copy(idx_hbm.at[wid], idx_v)
    pltpu.sync_copy(scat_hbm.at[wid], scat_v)

    zeros16 = jnp.zeros((16,), jnp.float32)

    @pl.loop(0, BLOCK)
    def _(r):
        @pl.loop(0, EMBED_DIM, step=16)
        def _(c):
            rows_v[0, r, pl.ds(c, 16)] = zeros16

    pltpu.sync_copy(rows_v.at[0],
                    acc_sh.at[pl.ds(sid * ROWS_PER_WORKER, ROWS_PER_WORKER)])

    def gather(b, slot):
        return pltpu.make_async_copy(
            table_hbm.at[idx_v.at[b]], rows_v.at[slot], gsems.at[slot])

    def scatter_start(b, slot):
        pltpu.async_copy(
            rows_v.at[slot], acc_sh.at[scat_v.at[b]], ssems.at[slot],
            add=True)

    def scatter_wait(b, slot):
        pltpu.make_async_copy(
            rows_v.at[slot], acc_sh.at[scat_v.at[b]], ssems.at[slot]).wait()

    gather(0, 0).start()
    gather(1, 1).start()

    @pl.loop(0, NUM_BLOCKS, step=2)
    def _(b):
        gather(b, 0).wait()
        scatter_start(b, 0)
        gather(b + 1, 1).wait()
        scatter_start(b + 1, 1)
        scatter_wait(b, 0)

        @pl.when(b + 2 < NUM_BLOCKS)
        def _():
            gather(b + 2, 0).start()

        scatter_wait(b + 1, 1)

        @pl.when(b + 3 < NUM_BLOCKS)
        def _():
            gather(b + 3, 1).start()

    pltpu.sync_copy(acc_sh.at[pl.ds(sid * ROWS_PER_WORKER, ROWS_PER_WORKER)],
                    out_hbm.at[pl.ds(wid * ROWS_PER_WORKER, ROWS_PER_WORKER)])


def _make_pool(rows):
    mesh = plsc.VectorSubcoreMesh(core_axis_name="c", subcore_axis_name="s")
    return pl.kernel(
        _sc_pool_kernel,
        out_type=jax.ShapeDtypeStruct((BATCH, EMBED_DIM), jnp.float32),
        mesh=mesh,
        compiler_params=pltpu.CompilerParams(use_tc_tiling_on_sc=False),
        scratch_types=[
            pltpu.VMEM((NUM_BLOCKS, BLOCK), jnp.int32),
            pltpu.VMEM((NUM_BLOCKS, BLOCK), jnp.int32),
            pltpu.VMEM((2, BLOCK, EMBED_DIM), jnp.float32),
            pltpu.VMEM_SHARED((ACC_ROWS, EMBED_DIM), jnp.float32),
            pltpu.SemaphoreType.DMA((2,)),
            pltpu.SemaphoreType.DMA((2,)),
        ],
    )


@jax.jit
def _pooled_halves(input_text, table):
    flat = input_text.reshape(NUM_WORKERS, NUM_BLOCKS, BLOCK)
    base = (jnp.arange(IDX_PER_WORKER, dtype=jnp.int32) // SEQ).reshape(
        1, NUM_BLOCKS, BLOCK)
    offs = ((jnp.arange(NUM_WORKERS, dtype=jnp.int32) // NUM_CORES)
            * ROWS_PER_WORKER).reshape(NUM_WORKERS, 1, 1)
    scat_real = base + offs
    junk = scat_real + CORE_ROWS
    trash = (flat & 127) * 2

    in_lo = flat < SPLIT
    idx_lo = jnp.where(in_lo, flat * 2, trash)
    scat_lo = jnp.where(in_lo, scat_real, junk)
    idx_hi = jnp.where(in_lo, trash, (flat - SPLIT) * 2)
    scat_hi = jnp.where(in_lo, junk, scat_real)

    table_t = table.T
    lin_lo = _transpose_pad(table_t, 0).reshape(2 * HALF_ROWS[0], EMBED_DIM)
    pooled_lo = _make_pool(HALF_ROWS[0])(idx_lo, scat_lo, lin_lo)
    lin_hi = _transpose_pad(table_t, 1).reshape(2 * HALF_ROWS[1], EMBED_DIM)
    pooled_hi = _make_pool(HALF_ROWS[1])(idx_hi, scat_hi, lin_hi)
    return pooled_lo, pooled_hi


def _mlp_kernel(xa_ref, xb_ref, len_ref, w1_ref, b1_ref, w2_ref, b2_ref,
                out_ref):
    x = (xa_ref[...] + xb_ref[...]) / len_ref[...]
    h = jnp.maximum(
        jnp.dot(x, w1_ref[...], preferred_element_type=jnp.float32)
        + b1_ref[...], 0.0)
    out_ref[...] = (
        jnp.dot(h, w2_ref[...], preferred_element_type=jnp.float32)
        + b2_ref[...])


@jax.jit
def _mlp(pooled_lo, pooled_hi, text_len, W1, b1, W2, b2):
    bm = 512
    n_hidden = W1.shape[1]
    n_classes = W2.shape[1]
    grid = (BATCH // bm,)
    return pl.pallas_call(
        _mlp_kernel,
        grid=grid,
        in_specs=[
            pl.BlockSpec((bm, EMBED_DIM), lambda i: (i, 0)),
            pl.BlockSpec((bm, EMBED_DIM), lambda i: (i, 0)),
            pl.BlockSpec((bm, 1), lambda i: (i, 0)),
            pl.BlockSpec((EMBED_DIM, n_hidden), lambda i: (0, 0)),
            pl.BlockSpec((1, n_hidden), lambda i: (0, 0)),
            pl.BlockSpec((n_hidden, n_classes), lambda i: (0, 0)),
            pl.BlockSpec((1, n_classes), lambda i: (0, 0)),
        ],
        out_specs=pl.BlockSpec((bm, n_classes), lambda i: (i, 0)),
        out_shape=jax.ShapeDtypeStruct((BATCH, n_classes), jnp.float32),
    )(pooled_lo, pooled_hi, text_len.reshape(BATCH, 1), W1,
      b1.reshape(1, n_hidden), W2, b2.reshape(1, n_classes))


def kernel(input_text, text_len, table, W1, b1, W2, b2):
    pooled_lo, pooled_hi = _pooled_halves(input_text, table)
    return _mlp(pooled_lo, pooled_hi, text_len, W1, b1, W2, b2)


# speedup vs baseline: 26.4223x; 1.3559x over previous
import jax
import jax.numpy as jnp
from jax import lax
from jax.experimental import pallas as pl
from jax.experimental.pallas import tpu as pltpu
from jax.experimental.pallas import tpu_sc as plsc

VOCAB = 1000000
EMBED_DIM = 64
PAD_DIM = 128
BATCH = 4096
SEQ = 200

NUM_CORES = 2
NUM_SUBCORES = 16
NUM_WORKERS = NUM_CORES * NUM_SUBCORES
IDX_PER_WORKER = BATCH * SEQ // NUM_WORKERS
ROWS_PER_WORKER = BATCH // NUM_WORKERS
BLOCK = 128
NUM_BLOCKS = IDX_PER_WORKER // BLOCK

TP_CHUNK = 4096


def _transpose_pad_kernel(in_ref, out_ref):
    xt = in_ref[...].T
    out_ref[...] = jnp.concatenate(
        [xt[:TP_CHUNK // 2], xt[TP_CHUNK // 2:]], axis=1)


@jax.jit
def _transpose_pad(table_t):
    grid = (pl.cdiv(VOCAB, TP_CHUNK),)
    return pl.pallas_call(
        _transpose_pad_kernel,
        grid=grid,
        in_specs=[pl.BlockSpec((EMBED_DIM, TP_CHUNK), lambda j: (0, j))],
        out_specs=pl.BlockSpec((TP_CHUNK // 2, PAD_DIM), lambda j: (j, 0)),
        out_shape=jax.ShapeDtypeStruct((VOCAB // 2, PAD_DIM), jnp.float32),
    )(table_t)


def _sc_pool_kernel(idx_hbm, scat_hbm, table_hbm, out_hbm,
                    idx_v, scat_v, rows_v, acc_sh, gsems, ssems):
    cid = lax.axis_index("c")
    sid = lax.axis_index("s")
    wid = sid * NUM_CORES + cid

    pltpu.sync_copy(idx_hbm.at[wid], idx_v)
    pltpu.sync_copy(scat_hbm.at[sid], scat_v)

    zeros16 = jnp.zeros((16,), jnp.float32)

    @pl.loop(0, BLOCK)
    def _(r):
        @pl.loop(0, EMBED_DIM, step=16)
        def _(c):
            rows_v[0, r, pl.ds(c, 16)] = zeros16

    pltpu.sync_copy(rows_v.at[0],
                    acc_sh.at[pl.ds(sid * ROWS_PER_WORKER, ROWS_PER_WORKER)])

    def gather(b, slot):
        return pltpu.make_async_copy(
            table_hbm.at[idx_v.at[b]], rows_v.at[slot], gsems.at[slot])

    def scatter_start(b, slot):
        pltpu.async_copy(
            rows_v.at[slot], acc_sh.at[scat_v.at[b]], ssems.at[slot],
            add=True)

    def scatter_wait(b, slot):
        pltpu.make_async_copy(
            rows_v.at[slot], acc_sh.at[scat_v.at[b]], ssems.at[slot]).wait()

    gather(0, 0).start()
    gather(1, 1).start()

    @pl.loop(0, NUM_BLOCKS, step=2)
    def _(b):
        gather(b, 0).wait()
        scatter_start(b, 0)
        gather(b + 1, 1).wait()
        scatter_start(b + 1, 1)
        scatter_wait(b, 0)

        @pl.when(b + 2 < NUM_BLOCKS)
        def _():
            gather(b + 2, 0).start()

        scatter_wait(b + 1, 1)

        @pl.when(b + 3 < NUM_BLOCKS)
        def _():
            gather(b + 3, 1).start()

    pltpu.sync_copy(acc_sh.at[pl.ds(sid * ROWS_PER_WORKER, ROWS_PER_WORKER)],
                    out_hbm.at[pl.ds(wid * ROWS_PER_WORKER, ROWS_PER_WORKER)])


@jax.jit
def _sc_pool(input_text, table):
    flat = input_text.reshape(NUM_WORKERS, NUM_BLOCKS, BLOCK)
    base = flat & ~(TP_CHUNK - 1)
    q = flat & (TP_CHUNK - 1)
    idx = base + ((q & (TP_CHUNK // 2 - 1)) << 1) + (q // (TP_CHUNK // 2))
    base = (jnp.arange(IDX_PER_WORKER, dtype=jnp.int32) // SEQ).reshape(
        1, NUM_BLOCKS, BLOCK)
    offs = (jnp.arange(NUM_SUBCORES, dtype=jnp.int32)
            * ROWS_PER_WORKER).reshape(NUM_SUBCORES, 1, 1)
    scat = base + offs

    table_lin = _transpose_pad(table.T).reshape(VOCAB, EMBED_DIM)

    mesh = plsc.VectorSubcoreMesh(core_axis_name="c", subcore_axis_name="s")
    pool = pl.kernel(
        _sc_pool_kernel,
        out_type=jax.ShapeDtypeStruct((BATCH, EMBED_DIM), jnp.float32),
        mesh=mesh,
        compiler_params=pltpu.CompilerParams(use_tc_tiling_on_sc=False),
        scratch_types=[
            pltpu.VMEM((NUM_BLOCKS, BLOCK), jnp.int32),
            pltpu.VMEM((NUM_BLOCKS, BLOCK), jnp.int32),
            pltpu.VMEM((2, BLOCK, EMBED_DIM), jnp.float32),
            pltpu.VMEM_SHARED((NUM_SUBCORES * ROWS_PER_WORKER, EMBED_DIM),
                              jnp.float32),
            pltpu.SemaphoreType.DMA((2,)),
            pltpu.SemaphoreType.DMA((2,)),
        ],
    )
    return pool(idx, scat, table_lin)


def _mlp_kernel(x_ref, len_ref, w1_ref, b1_ref, w2_ref, b2_ref, out_ref):
    x = x_ref[...] / len_ref[...]
    h = jnp.maximum(
        jnp.dot(x, w1_ref[...], preferred_element_type=jnp.float32)
        + b1_ref[...], 0.0)
    out_ref[...] = (
        jnp.dot(h, w2_ref[...], preferred_element_type=jnp.float32)
        + b2_ref[...])


@jax.jit
def _mlp(pooled, text_len, W1, b1, W2, b2):
    bm = 512
    n_hidden = W1.shape[1]
    n_classes = W2.shape[1]
    grid = (BATCH // bm,)
    return pl.pallas_call(
        _mlp_kernel,
        grid=grid,
        in_specs=[
            pl.BlockSpec((bm, EMBED_DIM), lambda i: (i, 0)),
            pl.BlockSpec((bm, 1), lambda i: (i, 0)),
            pl.BlockSpec((EMBED_DIM, n_hidden), lambda i: (0, 0)),
            pl.BlockSpec((1, n_hidden), lambda i: (0, 0)),
            pl.BlockSpec((n_hidden, n_classes), lambda i: (0, 0)),
            pl.BlockSpec((1, n_classes), lambda i: (0, 0)),
        ],
        out_specs=pl.BlockSpec((bm, n_classes), lambda i: (i, 0)),
        out_shape=jax.ShapeDtypeStruct((BATCH, n_classes), jnp.float32),
    )(pooled, text_len.reshape(BATCH, 1), W1, b1.reshape(1, n_hidden),
      W2, b2.reshape(1, n_classes))


def kernel(input_text, text_len, table, W1, b1, W2, b2):
    pooled = _sc_pool(input_text, table)
    return _mlp(pooled, text_len, W1, b1, W2, b2)


# speedup vs baseline: 30.3846x; 1.1500x over previous
import jax
import jax.numpy as jnp
from jax import lax
from jax.experimental import pallas as pl
from jax.experimental.pallas import tpu as pltpu
from jax.experimental.pallas import tpu_sc as plsc

VOCAB = 1000000
EMBED_DIM = 64
PAD_DIM = 128
BATCH = 4096
SEQ = 200

NUM_CORES = 2
NUM_SUBCORES = 16
NUM_WORKERS = NUM_CORES * NUM_SUBCORES
IDX_PER_WORKER = BATCH * SEQ // NUM_WORKERS
ROWS_PER_WORKER = BATCH // NUM_WORKERS
BLOCK = 128
NUM_BLOCKS = IDX_PER_WORKER // BLOCK

TP_CHUNK = 8192


def _transpose_pad_kernel(in_ref, out_ref):
    out_ref[:, :EMBED_DIM] = in_ref[...].T


@jax.jit
def _transpose_pad(table_t):
    grid = (pl.cdiv(VOCAB, TP_CHUNK),)
    return pl.pallas_call(
        _transpose_pad_kernel,
        grid=grid,
        in_specs=[pl.BlockSpec((EMBED_DIM, TP_CHUNK), lambda j: (0, j))],
        out_specs=pl.BlockSpec((TP_CHUNK, PAD_DIM), lambda j: (j, 0)),
        out_shape=jax.ShapeDtypeStruct((VOCAB, PAD_DIM), jnp.float32),
    )(table_t)


def _sc_pool_kernel(idx_hbm, scat_hbm, table_hbm, out_hbm,
                    idx_v, scat_v, rows_v, acc_sh, gsems, ssems):
    cid = lax.axis_index("c")
    sid = lax.axis_index("s")
    wid = sid * NUM_CORES + cid

    pltpu.sync_copy(idx_hbm.at[wid], idx_v)
    pltpu.sync_copy(scat_hbm.at[sid], scat_v)

    zeros16 = jnp.zeros((16,), jnp.float32)

    @pl.loop(0, BLOCK)
    def _(r):
        @pl.loop(0, EMBED_DIM, step=16)
        def _(c):
            rows_v[0, r, pl.ds(c, 16)] = zeros16

    pltpu.sync_copy(rows_v.at[0],
                    acc_sh.at[pl.ds(sid * ROWS_PER_WORKER, ROWS_PER_WORKER)])

    def gather(b, slot):
        return pltpu.make_async_copy(
            table_hbm.at[idx_v.at[b]], rows_v.at[slot], gsems.at[slot])

    def scatter_start(b, slot):
        pltpu.async_copy(
            rows_v.at[slot], acc_sh.at[scat_v.at[b]], ssems.at[slot],
            add=True)

    def scatter_wait(b, slot):
        pltpu.make_async_copy(
            rows_v.at[slot], acc_sh.at[scat_v.at[b]], ssems.at[slot]).wait()

    gather(0, 0).start()
    gather(1, 1).start()

    @pl.loop(0, NUM_BLOCKS, step=2)
    def _(b):
        gather(b, 0).wait()
        scatter_start(b, 0)
        gather(b + 1, 1).wait()
        scatter_start(b + 1, 1)
        scatter_wait(b, 0)

        @pl.when(b + 2 < NUM_BLOCKS)
        def _():
            gather(b + 2, 0).start()

        scatter_wait(b + 1, 1)

        @pl.when(b + 3 < NUM_BLOCKS)
        def _():
            gather(b + 3, 1).start()

    pltpu.sync_copy(acc_sh.at[pl.ds(sid * ROWS_PER_WORKER, ROWS_PER_WORKER)],
                    out_hbm.at[pl.ds(wid * ROWS_PER_WORKER, ROWS_PER_WORKER)])


@jax.jit
def _sc_pool(input_text, table):
    idx = input_text.reshape(NUM_WORKERS, NUM_BLOCKS, BLOCK) * 2
    base = (jnp.arange(IDX_PER_WORKER, dtype=jnp.int32) // SEQ).reshape(
        1, NUM_BLOCKS, BLOCK)
    offs = (jnp.arange(NUM_SUBCORES, dtype=jnp.int32)
            * ROWS_PER_WORKER).reshape(NUM_SUBCORES, 1, 1)
    scat = base + offs

    table_lin = _transpose_pad(table.T).reshape(2 * VOCAB, EMBED_DIM)

    mesh = plsc.VectorSubcoreMesh(core_axis_name="c", subcore_axis_name="s")
    pool = pl.kernel(
        _sc_pool_kernel,
        out_type=jax.ShapeDtypeStruct((BATCH, EMBED_DIM), jnp.float32),
        mesh=mesh,
        compiler_params=pltpu.CompilerParams(use_tc_tiling_on_sc=False),
        scratch_types=[
            pltpu.VMEM((NUM_BLOCKS, BLOCK), jnp.int32),
            pltpu.VMEM((NUM_BLOCKS, BLOCK), jnp.int32),
            pltpu.VMEM((2, BLOCK, EMBED_DIM), jnp.float32),
            pltpu.VMEM_SHARED((NUM_SUBCORES * ROWS_PER_WORKER, EMBED_DIM),
                              jnp.float32),
            pltpu.SemaphoreType.DMA((2,)),
            pltpu.SemaphoreType.DMA((2,)),
        ],
    )
    return pool(idx, scat, table_lin)


def _mlp_kernel(x_ref, len_ref, w1_ref, b1_ref, w2_ref, b2_ref, out_ref):
    x = x_ref[...] / len_ref[...]
    h = jnp.maximum(
        jnp.dot(x, w1_ref[...], preferred_element_type=jnp.float32)
        + b1_ref[...], 0.0)
    out_ref[...] = (
        jnp.dot(h, w2_ref[...], preferred_element_type=jnp.float32)
        + b2_ref[...])


@jax.jit
def _mlp(pooled, text_len, W1, b1, W2, b2):
    bm = 512
    n_hidden = W1.shape[1]
    n_classes = W2.shape[1]
    grid = (BATCH // bm,)
    return pl.pallas_call(
        _mlp_kernel,
        grid=grid,
        in_specs=[
            pl.BlockSpec((bm, EMBED_DIM), lambda i: (i, 0)),
            pl.BlockSpec((bm, 1), lambda i: (i, 0)),
            pl.BlockSpec((EMBED_DIM, n_hidden), lambda i: (0, 0)),
            pl.BlockSpec((1, n_hidden), lambda i: (0, 0)),
            pl.BlockSpec((n_hidden, n_classes), lambda i: (0, 0)),
            pl.BlockSpec((1, n_classes), lambda i: (0, 0)),
        ],
        out_specs=pl.BlockSpec((bm, n_classes), lambda i: (i, 0)),
        out_shape=jax.ShapeDtypeStruct((BATCH, n_classes), jnp.float32),
    )(pooled, text_len.reshape(BATCH, 1), W1, b1.reshape(1, n_hidden),
      W2, b2.reshape(1, n_classes))


def kernel(input_text, text_len, table, W1, b1, W2, b2):
    pooled = _sc_pool(input_text, table)
    return _mlp(pooled, text_len, W1, b1, W2, b2)


# speedup vs baseline: 31.2952x; 1.0300x over previous
import jax
import jax.numpy as jnp
from jax import lax
from jax.experimental import pallas as pl
from jax.experimental.pallas import tpu as pltpu
from jax.experimental.pallas import tpu_sc as plsc

VOCAB = 1000000
EMBED_DIM = 64
PAD_DIM = 128
BATCH = 4096
SEQ = 200

NUM_CORES = 2
NUM_SUBCORES = 16
NUM_WORKERS = NUM_CORES * NUM_SUBCORES
IDX_PER_WORKER = BATCH * SEQ // NUM_WORKERS
ROWS_PER_WORKER = BATCH // NUM_WORKERS
BLOCK = 128
NUM_BLOCKS = IDX_PER_WORKER // BLOCK

TP_CHUNK = 16384


def _transpose_pad_kernel(in_ref, out_ref):
    out_ref[:, :EMBED_DIM] = in_ref[...].T


@jax.jit
def _transpose_pad(table_t):
    grid = (pl.cdiv(VOCAB, TP_CHUNK),)
    return pl.pallas_call(
        _transpose_pad_kernel,
        grid=grid,
        in_specs=[pl.BlockSpec((EMBED_DIM, TP_CHUNK), lambda j: (0, j))],
        out_specs=pl.BlockSpec((TP_CHUNK, PAD_DIM), lambda j: (j, 0)),
        out_shape=jax.ShapeDtypeStruct((VOCAB, PAD_DIM), jnp.float32),
    )(table_t)


def _sc_pool_kernel(idx_hbm, scat_hbm, table_hbm, out_hbm,
                    idx_v, scat_v, rows_v, acc_sh, gsems, ssems):
    cid = lax.axis_index("c")
    sid = lax.axis_index("s")
    wid = sid * NUM_CORES + cid

    pltpu.sync_copy(idx_hbm.at[wid], idx_v)
    pltpu.sync_copy(scat_hbm.at[sid], scat_v)

    zeros16 = jnp.zeros((16,), jnp.float32)

    @pl.loop(0, BLOCK)
    def _(r):
        @pl.loop(0, EMBED_DIM, step=16)
        def _(c):
            rows_v[0, r, pl.ds(c, 16)] = zeros16

    pltpu.sync_copy(rows_v.at[0],
                    acc_sh.at[pl.ds(sid * ROWS_PER_WORKER, ROWS_PER_WORKER)])

    def gather(b, slot):
        return pltpu.make_async_copy(
            table_hbm.at[idx_v.at[b]], rows_v.at[slot], gsems.at[slot])

    def scatter_start(b, slot):
        pltpu.async_copy(
            rows_v.at[slot], acc_sh.at[scat_v.at[b]], ssems.at[slot],
            add=True)

    def scatter_wait(b, slot):
        pltpu.make_async_copy(
            rows_v.at[slot], acc_sh.at[scat_v.at[b]], ssems.at[slot]).wait()

    gather(0, 0).start()
    gather(1, 1).start()

    @pl.loop(0, NUM_BLOCKS, step=2)
    def _(b):
        gather(b, 0).wait()
        scatter_start(b, 0)
        gather(b + 1, 1).wait()
        scatter_start(b + 1, 1)
        scatter_wait(b, 0)

        @pl.when(b + 2 < NUM_BLOCKS)
        def _():
            gather(b + 2, 0).start()

        scatter_wait(b + 1, 1)

        @pl.when(b + 3 < NUM_BLOCKS)
        def _():
            gather(b + 3, 1).start()

    pltpu.sync_copy(acc_sh.at[pl.ds(sid * ROWS_PER_WORKER, ROWS_PER_WORKER)],
                    out_hbm.at[pl.ds(wid * ROWS_PER_WORKER, ROWS_PER_WORKER)])


@jax.jit
def _sc_pool(input_text, table):
    idx = input_text.reshape(NUM_WORKERS, NUM_BLOCKS, BLOCK) * 2
    base = (jnp.arange(IDX_PER_WORKER, dtype=jnp.int32) // SEQ).reshape(
        1, NUM_BLOCKS, BLOCK)
    offs = (jnp.arange(NUM_SUBCORES, dtype=jnp.int32)
            * ROWS_PER_WORKER).reshape(NUM_SUBCORES, 1, 1)
    scat = base + offs

    table_lin = _transpose_pad(table.T).reshape(2 * VOCAB, EMBED_DIM)

    mesh = plsc.VectorSubcoreMesh(core_axis_name="c", subcore_axis_name="s")
    pool = pl.kernel(
        _sc_pool_kernel,
        out_type=jax.ShapeDtypeStruct((BATCH, EMBED_DIM), jnp.float32),
        mesh=mesh,
        compiler_params=pltpu.CompilerParams(use_tc_tiling_on_sc=False),
        scratch_types=[
            pltpu.VMEM((NUM_BLOCKS, BLOCK), jnp.int32),
            pltpu.VMEM((NUM_BLOCKS, BLOCK), jnp.int32),
            pltpu.VMEM((2, BLOCK, EMBED_DIM), jnp.float32),
            pltpu.VMEM_SHARED((NUM_SUBCORES * ROWS_PER_WORKER, EMBED_DIM),
                              jnp.float32),
            pltpu.SemaphoreType.DMA((2,)),
            pltpu.SemaphoreType.DMA((2,)),
        ],
    )
    return pool(idx, scat, table_lin)


def _mlp_kernel(x_ref, len_ref, w1_ref, b1_ref, w2_ref, b2_ref, out_ref):
    x = x_ref[...] / len_ref[...]
    h = jnp.maximum(
        jnp.dot(x, w1_ref[...], preferred_element_type=jnp.float32)
        + b1_ref[...], 0.0)
    out_ref[...] = (
        jnp.dot(h, w2_ref[...], preferred_element_type=jnp.float32)
        + b2_ref[...])


@jax.jit
def _mlp(pooled, text_len, W1, b1, W2, b2):
    bm = 512
    n_hidden = W1.shape[1]
    n_classes = W2.shape[1]
    grid = (BATCH // bm,)
    return pl.pallas_call(
        _mlp_kernel,
        grid=grid,
        in_specs=[
            pl.BlockSpec((bm, EMBED_DIM), lambda i: (i, 0)),
            pl.BlockSpec((bm, 1), lambda i: (i, 0)),
            pl.BlockSpec((EMBED_DIM, n_hidden), lambda i: (0, 0)),
            pl.BlockSpec((1, n_hidden), lambda i: (0, 0)),
            pl.BlockSpec((n_hidden, n_classes), lambda i: (0, 0)),
            pl.BlockSpec((1, n_classes), lambda i: (0, 0)),
        ],
        out_specs=pl.BlockSpec((bm, n_classes), lambda i: (i, 0)),
        out_shape=jax.ShapeDtypeStruct((BATCH, n_classes), jnp.float32),
    )(pooled, text_len.reshape(BATCH, 1), W1, b1.reshape(1, n_hidden),
      W2, b2.reshape(1, n_classes))


def kernel(input_text, text_len, table, W1, b1, W2, b2):
    pooled = _sc_pool(input_text, table)
    return _mlp(pooled, text_len, W1, b1, W2, b2)


# speedup vs baseline: 31.5325x; 1.0076x over previous
import jax
import jax.numpy as jnp
from jax import lax
from jax.experimental import pallas as pl
from jax.experimental.pallas import tpu as pltpu
from jax.experimental.pallas import tpu_sc as plsc

VOCAB = 1000000
EMBED_DIM = 64
PAD_DIM = 128
BATCH = 4096
SEQ = 200

NUM_CORES = 2
NUM_SUBCORES = 16
NUM_WORKERS = NUM_CORES * NUM_SUBCORES
IDX_PER_WORKER = BATCH * SEQ // NUM_WORKERS
ROWS_PER_WORKER = BATCH // NUM_WORKERS
BLOCK = 128
NUM_BLOCKS = IDX_PER_WORKER // BLOCK

TP_CHUNK = 32768


def _transpose_pad_kernel(in_ref, out_ref):
    out_ref[:, :EMBED_DIM] = in_ref[...].T


@jax.jit
def _transpose_pad(table_t):
    grid = (pl.cdiv(VOCAB, TP_CHUNK),)
    return pl.pallas_call(
        _transpose_pad_kernel,
        grid=grid,
        in_specs=[pl.BlockSpec((EMBED_DIM, TP_CHUNK), lambda j: (0, j))],
        out_specs=pl.BlockSpec((TP_CHUNK, PAD_DIM), lambda j: (j, 0)),
        out_shape=jax.ShapeDtypeStruct((VOCAB, PAD_DIM), jnp.float32),
    )(table_t)


def _sc_pool_kernel(idx_hbm, scat_hbm, table_hbm, out_hbm,
                    idx_v, scat_v, rows_v, acc_sh, gsems, ssems):
    cid = lax.axis_index("c")
    sid = lax.axis_index("s")
    wid = sid * NUM_CORES + cid

    pltpu.sync_copy(idx_hbm.at[wid], idx_v)
    pltpu.sync_copy(scat_hbm.at[sid], scat_v)

    zeros16 = jnp.zeros((16,), jnp.float32)

    @pl.loop(0, BLOCK)
    def _(r):
        @pl.loop(0, EMBED_DIM, step=16)
        def _(c):
            rows_v[0, r, pl.ds(c, 16)] = zeros16

    pltpu.sync_copy(rows_v.at[0],
                    acc_sh.at[pl.ds(sid * ROWS_PER_WORKER, ROWS_PER_WORKER)])

    def gather(b, slot):
        return pltpu.make_async_copy(
            table_hbm.at[idx_v.at[b]], rows_v.at[slot], gsems.at[slot])

    def scatter_start(b, slot):
        pltpu.async_copy(
            rows_v.at[slot], acc_sh.at[scat_v.at[b]], ssems.at[slot],
            add=True)

    def scatter_wait(b, slot):
        pltpu.make_async_copy(
            rows_v.at[slot], acc_sh.at[scat_v.at[b]], ssems.at[slot]).wait()

    gather(0, 0).start()
    gather(1, 1).start()

    @pl.loop(0, NUM_BLOCKS, step=2)
    def _(b):
        gather(b, 0).wait()
        scatter_start(b, 0)
        gather(b + 1, 1).wait()
        scatter_start(b + 1, 1)
        scatter_wait(b, 0)

        @pl.when(b + 2 < NUM_BLOCKS)
        def _():
            gather(b + 2, 0).start()

        scatter_wait(b + 1, 1)

        @pl.when(b + 3 < NUM_BLOCKS)
        def _():
            gather(b + 3, 1).start()

    pltpu.sync_copy(acc_sh.at[pl.ds(sid * ROWS_PER_WORKER, ROWS_PER_WORKER)],
                    out_hbm.at[pl.ds(wid * ROWS_PER_WORKER, ROWS_PER_WORKER)])


@jax.jit
def _sc_pool(input_text, table):
    idx = input_text.reshape(NUM_WORKERS, NUM_BLOCKS, BLOCK) * 2
    base = (jnp.arange(IDX_PER_WORKER, dtype=jnp.int32) // SEQ).reshape(
        1, NUM_BLOCKS, BLOCK)
    offs = (jnp.arange(NUM_SUBCORES, dtype=jnp.int32)
            * ROWS_PER_WORKER).reshape(NUM_SUBCORES, 1, 1)
    scat = base + offs

    table_lin = _transpose_pad(table.T).reshape(2 * VOCAB, EMBED_DIM)

    mesh = plsc.VectorSubcoreMesh(core_axis_name="c", subcore_axis_name="s")
    pool = pl.kernel(
        _sc_pool_kernel,
        out_type=jax.ShapeDtypeStruct((BATCH, EMBED_DIM), jnp.float32),
        mesh=mesh,
        compiler_params=pltpu.CompilerParams(use_tc_tiling_on_sc=False),
        scratch_types=[
            pltpu.VMEM((NUM_BLOCKS, BLOCK), jnp.int32),
            pltpu.VMEM((NUM_BLOCKS, BLOCK), jnp.int32),
            pltpu.VMEM((2, BLOCK, EMBED_DIM), jnp.float32),
            pltpu.VMEM_SHARED((NUM_SUBCORES * ROWS_PER_WORKER, EMBED_DIM),
                              jnp.float32),
            pltpu.SemaphoreType.DMA((2,)),
            pltpu.SemaphoreType.DMA((2,)),
        ],
    )
    return pool(idx, scat, table_lin)


def _mlp_kernel(x_ref, len_ref, w1_ref, b1_ref, w2_ref, b2_ref, out_ref):
    x = x_ref[...] / len_ref[...]
    h = jnp.maximum(
        jnp.dot(x, w1_ref[...], preferred_element_type=jnp.float32)
        + b1_ref[...], 0.0)
    out_ref[...] = (
        jnp.dot(h, w2_ref[...], preferred_element_type=jnp.float32)
        + b2_ref[...])


@jax.jit
def _mlp(pooled, text_len, W1, b1, W2, b2):
    bm = 512
    n_hidden = W1.shape[1]
    n_classes = W2.shape[1]
    grid = (BATCH // bm,)
    return pl.pallas_call(
        _mlp_kernel,
        grid=grid,
        in_specs=[
            pl.BlockSpec((bm, EMBED_DIM), lambda i: (i, 0)),
            pl.BlockSpec((bm, 1), lambda i: (i, 0)),
            pl.BlockSpec((EMBED_DIM, n_hidden), lambda i: (0, 0)),
            pl.BlockSpec((1, n_hidden), lambda i: (0, 0)),
            pl.BlockSpec((n_hidden, n_classes), lambda i: (0, 0)),
            pl.BlockSpec((1, n_classes), lambda i: (0, 0)),
        ],
        out_specs=pl.BlockSpec((bm, n_classes), lambda i: (i, 0)),
        out_shape=jax.ShapeDtypeStruct((BATCH, n_classes), jnp.float32),
    )(pooled, text_len.reshape(BATCH, 1), W1, b1.reshape(1, n_hidden),
      W2, b2.reshape(1, n_classes))


def kernel(input_text, text_len, table, W1, b1, W2, b2):
    pooled = _sc_pool(input_text, table)
    return _mlp(pooled, text_len, W1, b1, W2, b2)


# speedup vs baseline: 35.5655x; 1.1279x over previous
import jax
import jax.numpy as jnp
from jax import lax
from jax.experimental import pallas as pl
from jax.experimental.pallas import tpu as pltpu
from jax.experimental.pallas import tpu_sc as plsc

VOCAB = 1000000
EMBED_DIM = 64
PAD_DIM = 128
BATCH = 4096
SEQ = 200

NUM_CORES = 2
NUM_SUBCORES = 16
NUM_WORKERS = NUM_CORES * NUM_SUBCORES
IDX_PER_WORKER = BATCH * SEQ // NUM_WORKERS
ROWS_PER_WORKER = BATCH // NUM_WORKERS
BLOCK = 128
NUM_BLOCKS = IDX_PER_WORKER // BLOCK
NSLOTS = 4

TP_CHUNK = 32768


def _transpose_pad_kernel(in_ref, out_ref):
    out_ref[:, :EMBED_DIM] = in_ref[...].T


@jax.jit
def _transpose_pad(table_t):
    grid = (pl.cdiv(VOCAB, TP_CHUNK),)
    return pl.pallas_call(
        _transpose_pad_kernel,
        grid=grid,
        in_specs=[pl.BlockSpec((EMBED_DIM, TP_CHUNK), lambda j: (0, j))],
        out_specs=pl.BlockSpec((TP_CHUNK, PAD_DIM), lambda j: (j, 0)),
        out_shape=jax.ShapeDtypeStruct((VOCAB, PAD_DIM), jnp.float32),
    )(table_t)


def _sc_pool_kernel(idx_hbm, scat_hbm, table_hbm, out_hbm,
                    idx_v, scat_v, rows_v, acc_sh, gsems, ssems):
    cid = lax.axis_index("c")
    sid = lax.axis_index("s")
    wid = sid * NUM_CORES + cid

    pltpu.sync_copy(idx_hbm.at[wid], idx_v)
    pltpu.sync_copy(scat_hbm.at[sid], scat_v)

    zeros16 = jnp.zeros((16,), jnp.float32)

    @pl.loop(0, BLOCK)
    def _(r):
        @pl.loop(0, EMBED_DIM, step=16)
        def _(c):
            rows_v[0, r, pl.ds(c, 16)] = zeros16

    pltpu.sync_copy(rows_v.at[0],
                    acc_sh.at[pl.ds(sid * ROWS_PER_WORKER, ROWS_PER_WORKER)])

    def gather(b, slot):
        return pltpu.make_async_copy(
            table_hbm.at[idx_v.at[b]], rows_v.at[slot], gsems.at[slot])

    def scatter_start(b, slot):
        pltpu.async_copy(
            rows_v.at[slot], acc_sh.at[scat_v.at[b]], ssems.at[slot],
            add=True)

    def scatter_wait(b, slot):
        pltpu.make_async_copy(
            rows_v.at[slot], acc_sh.at[scat_v.at[b]], ssems.at[slot]).wait()

    for s in range(NSLOTS):
        gather(s, s).start()

    @pl.loop(0, NUM_BLOCKS, step=NSLOTS)
    def _(b):
        for s in range(NSLOTS):
            gather(b + s, s).wait()
            scatter_start(b + s, s)
        for s in range(NSLOTS):
            scatter_wait(b + s, s)

            @pl.when(b + NSLOTS + s < NUM_BLOCKS)
            def _(s=s):
                gather(b + NSLOTS + s, s).start()

    pltpu.sync_copy(acc_sh.at[pl.ds(sid * ROWS_PER_WORKER, ROWS_PER_WORKER)],
                    out_hbm.at[pl.ds(wid * ROWS_PER_WORKER, ROWS_PER_WORKER)])


@jax.jit
def _sc_pool(input_text, table):
    idx = input_text.reshape(NUM_WORKERS, NUM_BLOCKS, BLOCK) * 2
    base = (jnp.arange(IDX_PER_WORKER, dtype=jnp.int32) // SEQ).reshape(
        1, NUM_BLOCKS, BLOCK)
    offs = (jnp.arange(NUM_SUBCORES, dtype=jnp.int32)
            * ROWS_PER_WORKER).reshape(NUM_SUBCORES, 1, 1)
    scat = base + offs

    table_lin = _transpose_pad(table.T).reshape(2 * VOCAB, EMBED_DIM)

    mesh = plsc.VectorSubcoreMesh(core_axis_name="c", subcore_axis_name="s")
    pool = pl.kernel(
        _sc_pool_kernel,
        out_type=jax.ShapeDtypeStruct((BATCH, EMBED_DIM), jnp.float32),
        mesh=mesh,
        compiler_params=pltpu.CompilerParams(use_tc_tiling_on_sc=False),
        scratch_types=[
            pltpu.VMEM((NUM_BLOCKS, BLOCK), jnp.int32),
            pltpu.VMEM((NUM_BLOCKS, BLOCK), jnp.int32),
            pltpu.VMEM((NSLOTS, BLOCK, EMBED_DIM), jnp.float32),
            pltpu.VMEM_SHARED((NUM_SUBCORES * ROWS_PER_WORKER, EMBED_DIM),
                              jnp.float32),
            pltpu.SemaphoreType.DMA((NSLOTS,)),
            pltpu.SemaphoreType.DMA((NSLOTS,)),
        ],
    )
    return pool(idx, scat, table_lin)


def _mlp_kernel(x_ref, len_ref, w1_ref, b1_ref, w2_ref, b2_ref, out_ref):
    x = x_ref[...] / len_ref[...]
    h = jnp.maximum(
        jnp.dot(x, w1_ref[...], preferred_element_type=jnp.float32)
        + b1_ref[...], 0.0)
    out_ref[...] = (
        jnp.dot(h, w2_ref[...], preferred_element_type=jnp.float32)
        + b2_ref[...])


@jax.jit
def _mlp(pooled, text_len, W1, b1, W2, b2):
    bm = 512
    n_hidden = W1.shape[1]
    n_classes = W2.shape[1]
    grid = (BATCH // bm,)
    return pl.pallas_call(
        _mlp_kernel,
        grid=grid,
        in_specs=[
            pl.BlockSpec((bm, EMBED_DIM), lambda i: (i, 0)),
            pl.BlockSpec((bm, 1), lambda i: (i, 0)),
            pl.BlockSpec((EMBED_DIM, n_hidden), lambda i: (0, 0)),
            pl.BlockSpec((1, n_hidden), lambda i: (0, 0)),
            pl.BlockSpec((n_hidden, n_classes), lambda i: (0, 0)),
            pl.BlockSpec((1, n_classes), lambda i: (0, 0)),
        ],
        out_specs=pl.BlockSpec((bm, n_classes), lambda i: (i, 0)),
        out_shape=jax.ShapeDtypeStruct((BATCH, n_classes), jnp.float32),
    )(pooled, text_len.reshape(BATCH, 1), W1, b1.reshape(1, n_hidden),
      W2, b2.reshape(1, n_classes))


def kernel(input_text, text_len, table, W1, b1, W2, b2):
    pooled = _sc_pool(input_text, table)
    return _mlp(pooled, text_len, W1, b1, W2, b2)


# speedup vs baseline: 36.1359x; 1.0160x over previous
import jax
import jax.numpy as jnp
from jax import lax
from jax.experimental import pallas as pl
from jax.experimental.pallas import tpu as pltpu
from jax.experimental.pallas import tpu_sc as plsc

VOCAB = 1000000
EMBED_DIM = 64
PAD_DIM = 128
BATCH = 4096
SEQ = 200

NUM_CORES = 2
NUM_SUBCORES = 16
NUM_WORKERS = NUM_CORES * NUM_SUBCORES
IDX_PER_WORKER = BATCH * SEQ // NUM_WORKERS
ROWS_PER_WORKER = BATCH // NUM_WORKERS
BLOCK = 128
NUM_BLOCKS = IDX_PER_WORKER // BLOCK
NSLOTS = 4

TP_CHUNK = 32768


def _transpose_pad_kernel(in_ref, out_ref):
    out_ref[:, :EMBED_DIM] = in_ref[...].T


@jax.jit
def _transpose_pad(table_t):
    grid = (pl.cdiv(VOCAB, TP_CHUNK),)
    return pl.pallas_call(
        _transpose_pad_kernel,
        grid=grid,
        in_specs=[pl.BlockSpec((EMBED_DIM, TP_CHUNK), lambda j: (0, j))],
        out_specs=pl.BlockSpec((TP_CHUNK, PAD_DIM), lambda j: (j, 0)),
        out_shape=jax.ShapeDtypeStruct((VOCAB, PAD_DIM), jnp.float32),
    )(table_t)


def _sc_pool_kernel(idx_hbm, scat_hbm, table_hbm, out_hbm,
                    idx_v, scat_v, rows_v, acc_sh, gsems, ssems):
    cid = lax.axis_index("c")
    sid = lax.axis_index("s")
    wid = sid * NUM_CORES + cid

    pltpu.sync_copy(idx_hbm.at[wid], idx_v)
    pltpu.sync_copy(scat_hbm.at[sid], scat_v)

    zeros16 = jnp.zeros((16,), jnp.float32)

    @pl.loop(0, BLOCK)
    def _(r):
        @pl.loop(0, EMBED_DIM, step=16)
        def _(c):
            rows_v[0, r, pl.ds(c, 16)] = zeros16

    pltpu.sync_copy(rows_v.at[0],
                    acc_sh.at[pl.ds(sid * ROWS_PER_WORKER, ROWS_PER_WORKER)])

    def gather(b, slot):
        return pltpu.make_async_copy(
            table_hbm.at[idx_v.at[b]], rows_v.at[slot], gsems.at[slot])

    def scatter_start(b, slot):
        pltpu.async_copy(
            rows_v.at[slot], acc_sh.at[scat_v.at[b]], ssems.at[slot],
            add=True)

    def scatter_wait(b, slot):
        pltpu.make_async_copy(
            rows_v.at[slot], acc_sh.at[scat_v.at[b]], ssems.at[slot]).wait()

    for s in range(NSLOTS):
        gather(s, s).start()

    @pl.loop(0, NUM_BLOCKS, step=NSLOTS)
    def _(b):
        gather(b, 0).wait()
        scatter_start(b, 0)
        gather(b + 1, 1).wait()
        scatter_start(b + 1, 1)
        for s in range(2, NSLOTS + 2):
            scatter_wait(b + s - 2, s - 2)

            @pl.when(b + NSLOTS + s - 2 < NUM_BLOCKS)
            def _(s=s):
                gather(b + NSLOTS + s - 2, s - 2).start()

            if s < NSLOTS:
                gather(b + s, s).wait()
                scatter_start(b + s, s)

    pltpu.sync_copy(acc_sh.at[pl.ds(sid * ROWS_PER_WORKER, ROWS_PER_WORKER)],
                    out_hbm.at[pl.ds(wid * ROWS_PER_WORKER, ROWS_PER_WORKER)])


@jax.jit
def _sc_pool(input_text, table):
    idx = input_text.reshape(NUM_WORKERS, NUM_BLOCKS, BLOCK) * 2
    base = (jnp.arange(IDX_PER_WORKER, dtype=jnp.int32) // SEQ).reshape(
        1, NUM_BLOCKS, BLOCK)
    offs = (jnp.arange(NUM_SUBCORES, dtype=jnp.int32)
            * ROWS_PER_WORKER).reshape(NUM_SUBCORES, 1, 1)
    scat = base + offs

    table_lin = _transpose_pad(table.T).reshape(2 * VOCAB, EMBED_DIM)

    mesh = plsc.VectorSubcoreMesh(core_axis_name="c", subcore_axis_name="s")
    pool = pl.kernel(
        _sc_pool_kernel,
        out_type=jax.ShapeDtypeStruct((BATCH, EMBED_DIM), jnp.float32),
        mesh=mesh,
        compiler_params=pltpu.CompilerParams(use_tc_tiling_on_sc=False),
        scratch_types=[
            pltpu.VMEM((NUM_BLOCKS, BLOCK), jnp.int32),
            pltpu.VMEM((NUM_BLOCKS, BLOCK), jnp.int32),
            pltpu.VMEM((NSLOTS, BLOCK, EMBED_DIM), jnp.float32),
            pltpu.VMEM_SHARED((NUM_SUBCORES * ROWS_PER_WORKER, EMBED_DIM),
                              jnp.float32),
            pltpu.SemaphoreType.DMA((NSLOTS,)),
            pltpu.SemaphoreType.DMA((NSLOTS,)),
        ],
    )
    return pool(idx, scat, table_lin)


def _mlp_kernel(x_ref, len_ref, w1_ref, b1_ref, w2_ref, b2_ref, out_ref):
    x = x_ref[...] / len_ref[...]
    h = jnp.maximum(
        jnp.dot(x, w1_ref[...], preferred_element_type=jnp.float32)
        + b1_ref[...], 0.0)
    out_ref[...] = (
        jnp.dot(h, w2_ref[...], preferred_element_type=jnp.float32)
        + b2_ref[...])


@jax.jit
def _mlp(pooled, text_len, W1, b1, W2, b2):
    bm = 512
    n_hidden = W1.shape[1]
    n_classes = W2.shape[1]
    grid = (BATCH // bm,)
    return pl.pallas_call(
        _mlp_kernel,
        grid=grid,
        in_specs=[
            pl.BlockSpec((bm, EMBED_DIM), lambda i: (i, 0)),
            pl.BlockSpec((bm, 1), lambda i: (i, 0)),
            pl.BlockSpec((EMBED_DIM, n_hidden), lambda i: (0, 0)),
            pl.BlockSpec((1, n_hidden), lambda i: (0, 0)),
            pl.BlockSpec((n_hidden, n_classes), lambda i: (0, 0)),
            pl.BlockSpec((1, n_classes), lambda i: (0, 0)),
        ],
        out_specs=pl.BlockSpec((bm, n_classes), lambda i: (i, 0)),
        out_shape=jax.ShapeDtypeStruct((BATCH, n_classes), jnp.float32),
    )(pooled, text_len.reshape(BATCH, 1), W1, b1.reshape(1, n_hidden),
      W2, b2.reshape(1, n_classes))


def kernel(input_text, text_len, table, W1, b1, W2, b2):
    pooled = _sc_pool(input_text, table)
    return _mlp(pooled, text_len, W1, b1, W2, b2)
